# Initial kernel scaffold; baseline (speedup 1.0000x reference)
import jax
import jax.numpy as jnp
from jax.experimental import pallas as pl


def kernel(edge_index, edge_type, embeddings, rgcn_weights, rgcn_biases):
    raise NotImplementedError("write your pallas kernel here")


# speedup vs baseline: 15.6676x; 15.6676x over previous
import functools

import jax
import jax.numpy as jnp
from jax import lax
from jax.experimental import pallas as pl
from jax.experimental.pallas import tpu as pltpu
from jax.experimental.pallas import tpu_sc as plsc

N = 10000
D = 128
NREL = 8
P = 2 * NREL
E = 320000
EI = 2 * E

NC = 2
NS = 16
NW = NC * NS
EPT = EI // NW
G = 80
NB = 5
NG = EPT // G
NBLK = NG // NB
NPT = N // NS

_mesh = plsc.VectorSubcoreMesh(core_axis_name="c", subcore_axis_name="s")
_sc_params = pltpu.CompilerParams(use_tc_tiling_on_sc=False,
                                  needs_layout_passes=False)


@functools.partial(
    pl.kernel,
    out_type=jax.ShapeDtypeStruct((NC, N, P), jnp.float32),
    mesh=_mesh,
    scratch_types=[
        pltpu.VMEM((NB, G), jnp.int32),
        pltpu.VMEM((NB, G), jnp.int32),
        pltpu.VMEM((G, P), jnp.float32),
        pltpu.VMEM_SHARED((N, P), jnp.float32),
    ],
    compiler_params=_sc_params,
)
def _sc_counts(dst_hbm, p_hbm, z_hbm, out_hbm, dstv, pv, ohv, cnt_sp):
    c = lax.axis_index("c")
    s = lax.axis_index("s")
    wid = c * NS + s
    pltpu.sync_copy(z_hbm.at[pl.ds(s * NPT, NPT)], cnt_sp.at[pl.ds(s * NPT, NPT)])
    zeros16 = jnp.zeros((16,), jnp.float32)
    ones16 = jnp.full((16,), 1.0, jnp.float32)
    for i in range(G):
        ohv[i, :] = zeros16
    plsc.subcore_barrier()
    iota16 = lax.iota(jnp.int32, 16)

    def blk_body(blk, _):
        row0 = wid * NG + blk * NB
        pltpu.sync_copy(dst_hbm.at[pl.ds(row0, NB)], dstv)
        pltpu.sync_copy(p_hbm.at[pl.ds(row0, NB)], pv)
        for b in range(NB):
            for k in range(G // 16):
                pk = pv[b, pl.ds(k * 16, 16)]
                plsc.store_scatter(ohv, [iota16 + k * 16, pk], ones16)
            pltpu.sync_copy(ohv, cnt_sp.at[dstv.at[b]], add=True)
            for k in range(G // 16):
                pk = pv[b, pl.ds(k * 16, 16)]
                plsc.store_scatter(ohv, [iota16 + k * 16, pk], zeros16)
        return 0

    lax.fori_loop(0, NBLK, blk_body, 0)
    plsc.subcore_barrier()
    pltpu.sync_copy(cnt_sp.at[pl.ds(s * NPT, NPT)],
                    out_hbm.at[c, pl.ds(s * NPT, NPT)])


@functools.partial(
    pl.kernel,
    out_type=jax.ShapeDtypeStruct((NC, N, D), jnp.float32),
    mesh=_mesh,
    scratch_types=[
        pltpu.VMEM((NB, G), jnp.int32),
        pltpu.VMEM((NB, G), jnp.int32),
        pltpu.VMEM((NB, G), jnp.int32),
        pltpu.VMEM((G, P), jnp.float32),
        pltpu.VMEM((G, D), jnp.float32),
        pltpu.VMEM((G + 16,), jnp.float32),
        pltpu.VMEM_SHARED((N, D), jnp.float32),
        pltpu.SemaphoreType.DMA,
        pltpu.SemaphoreType.DMA,
    ],
    compiler_params=_sc_params,
)
def _sc_edge(t_hbm, beta_hbm, tg_hbm, dst_hbm, p_hbm, z_hbm, out_hbm,
             tgv, dstv, pv, brows, trows, sv, acc_sp, sem_t, sem_b):
    c = lax.axis_index("c")
    s = lax.axis_index("s")
    wid = c * NS + s
    pltpu.sync_copy(z_hbm.at[pl.ds(s * NPT, NPT)], acc_sp.at[pl.ds(s * NPT, NPT)])
    plsc.subcore_barrier()
    iota16 = lax.iota(jnp.int32, 16)

    def blk_body(blk, _):
        row0 = wid * NG + blk * NB
        pltpu.sync_copy(tg_hbm.at[pl.ds(row0, NB)], tgv)
        pltpu.sync_copy(dst_hbm.at[pl.ds(row0, NB)], dstv)
        pltpu.sync_copy(p_hbm.at[pl.ds(row0, NB)], pv)
        for b in range(NB):
            cp_t = pltpu.async_copy(t_hbm.at[tgv.at[b]], trows, sem_t)
            cp_b = pltpu.async_copy(beta_hbm.at[dstv.at[b]], brows, sem_b)
            cp_b.wait()
            cp_t.wait()
            for k in range(G // 16):
                pk = pv[b, pl.ds(k * 16, 16)]
                sv[pl.ds(k * 16, 16)] = plsc.load_gather(
                    brows, [iota16 + k * 16, pk])

            def scale_body(j, _):
                val = sv[pl.ds(j, 16)][0]
                for ci in range(D // 16):
                    sl = pl.ds(ci * 16, 16)
                    trows[j, sl] = trows[j, sl] * val
                return 0

            lax.fori_loop(0, G, scale_body, 0)
            pltpu.sync_copy(trows, acc_sp.at[dstv.at[b]], add=True)
        return 0

    lax.fori_loop(0, NBLK, blk_body, 0)
    plsc.subcore_barrier()
    pltpu.sync_copy(acc_sp.at[pl.ds(s * NPT, NPT)],
                    out_hbm.at[c, pl.ds(s * NPT, NPT)])


_RB = 1000
_GRID = N // _RB


def _beta_body(cnt_ref, tri_ref, beta_ref, bc_ref):
    cnt = cnt_ref[0] + cnt_ref[1]
    f = 2.0 / jnp.maximum(cnt, 1.0)
    beta = jnp.exp(jnp.dot(jnp.log(f), tri_ref[...],
                           preferred_element_type=jnp.float32))
    beta_ref[...] = beta
    bc_ref[...] = beta * cnt


def _matmul_body(x_ref, w_ref, o_ref):
    o_ref[...] = jnp.dot(x_ref[...], w_ref[...],
                         preferred_element_type=jnp.float32)


def _combine_body(x_ref, ws_ref, bs_ref, beta_ref, bc_ref, bm_ref, acc_ref,
                  o_ref, *, relu):
    h = jnp.dot(x_ref[...], ws_ref[...],
                preferred_element_type=jnp.float32) + bs_ref[...]
    h = beta_ref[:, 0:1] * h
    h = h + acc_ref[0] + acc_ref[1]
    h = h + jnp.dot(bc_ref[...], bm_ref[...],
                    preferred_element_type=jnp.float32)
    if relu:
        h = jnp.maximum(h, 0.0)
    o_ref[...] = h


def _beta_call(cnt2, tri):
    return pl.pallas_call(
        _beta_body,
        grid=(_GRID,),
        in_specs=[
            pl.BlockSpec((NC, _RB, P), lambda i: (0, i, 0)),
            pl.BlockSpec((P, P), lambda i: (0, 0)),
        ],
        out_specs=[
            pl.BlockSpec((_RB, P), lambda i: (i, 0)),
            pl.BlockSpec((_RB, P), lambda i: (i, 0)),
        ],
        out_shape=[
            jax.ShapeDtypeStruct((N, P), jnp.float32),
            jax.ShapeDtypeStruct((N, P), jnp.float32),
        ],
    )(cnt2, tri)


def _matmul_call(x, w):
    k = w.shape[1]
    return pl.pallas_call(
        _matmul_body,
        grid=(_GRID,),
        in_specs=[
            pl.BlockSpec((_RB, D), lambda i: (i, 0)),
            pl.BlockSpec((D, k), lambda i: (0, 0)),
        ],
        out_specs=pl.BlockSpec((_RB, k), lambda i: (i, 0)),
        out_shape=jax.ShapeDtypeStruct((N, k), jnp.float32),
    )(x, w)


def _combine_call(x, wself, bself, beta, bc, bmat, acc2, relu):
    return pl.pallas_call(
        functools.partial(_combine_body, relu=relu),
        grid=(_GRID,),
        in_specs=[
            pl.BlockSpec((_RB, D), lambda i: (i, 0)),
            pl.BlockSpec((D, D), lambda i: (0, 0)),
            pl.BlockSpec((1, D), lambda i: (0, 0)),
            pl.BlockSpec((_RB, P), lambda i: (i, 0)),
            pl.BlockSpec((_RB, P), lambda i: (i, 0)),
            pl.BlockSpec((P, D), lambda i: (0, 0)),
            pl.BlockSpec((NC, _RB, D), lambda i: (0, i, 0)),
        ],
        out_specs=pl.BlockSpec((_RB, D), lambda i: (i, 0)),
        out_shape=jax.ShapeDtypeStruct((N, D), jnp.float32),
    )(x, wself, bself, beta, bc, bmat, acc2)


def kernel(edge_index, edge_type, embeddings, rgcn_weights, rgcn_biases):
    et = edge_type.astype(jnp.int32)
    dst = jnp.concatenate([edge_index[0], edge_index[1]]).astype(jnp.int32)
    src = jnp.concatenate([edge_index[1], edge_index[0]]).astype(jnp.int32)
    p = jnp.concatenate([2 * et, 2 * et + 1])
    tg = src * P + p
    dst2 = dst.reshape(EI // G, G)
    p2 = p.reshape(EI // G, G)
    tg2 = tg.reshape(EI // G, G)
    z16 = jnp.zeros((N, P), jnp.float32)
    z128 = jnp.zeros((N, D), jnp.float32)
    tri = jnp.triu(jnp.ones((P, P), jnp.float32)).T

    cnt2 = _sc_counts(dst2, p2, z16)
    beta, bc = _beta_call(cnt2, tri)

    perm = jnp.array([(q // 2) + NREL * (q % 2) for q in range(P)])
    output = embeddings
    for l in range(2):
        wall = jnp.transpose(rgcn_weights[l][perm], (1, 0, 2)).reshape(D, P * D)
        bmat = rgcn_biases[l][perm]
        t = _matmul_call(output, wall).reshape(N * P, D)
        acc2 = _sc_edge(t, beta, tg2, dst2, p2, z128)
        output = _combine_call(output, rgcn_weights[l, -1],
                               rgcn_biases[l, -1].reshape(1, D),
                               beta, bc, bmat, acc2, relu=(l == 0))
    return output


# speedup vs baseline: 15.7890x; 1.0078x over previous
import functools

import jax
import jax.numpy as jnp
from jax import lax
from jax.experimental import pallas as pl
from jax.experimental.pallas import tpu as pltpu
from jax.experimental.pallas import tpu_sc as plsc

N = 10000
D = 128
NREL = 8
P = 2 * NREL
E = 320000
EI = 2 * E

NC = 2
NS = 16
NW = NC * NS
G = 32
NPT = N // NS

_mesh = plsc.VectorSubcoreMesh(core_axis_name="c", subcore_axis_name="s")
_sc_params = pltpu.CompilerParams(use_tc_tiling_on_sc=False,
                                  needs_layout_passes=False)


EPC = E // NS
NR = EPC // G
RING = 5
AHEAD = 3
SB = 25
NSB = NR // SB
GK = G // 16


@functools.partial(
    pl.kernel,
    out_type=jax.ShapeDtypeStruct((NC, N, P), jnp.float32),
    mesh=_mesh,
    scratch_types=[
        pltpu.VMEM((NR, G), jnp.int32),
        pltpu.VMEM((NR, G), jnp.int32),
        pltpu.VMEM((G, P), jnp.float32),
        pltpu.VMEM_SHARED((N, P), jnp.float32),
    ],
    compiler_params=_sc_params,
)
def _sc_counts(ei_hbm, et_hbm, z_hbm, out_hbm, dstv, pv, ohv, cnt_sp):
    c = lax.axis_index("c")
    s = lax.axis_index("s")
    pltpu.sync_copy(z_hbm.at[pl.ds(s * NPT, NPT)], cnt_sp.at[pl.ds(s * NPT, NPT)])
    pltpu.sync_copy(ei_hbm.at[c, pl.ds(s * NR, NR)], dstv)
    pltpu.sync_copy(et_hbm.at[pl.ds(s * NR, NR)], pv)

    def cvt(i, _):
        row = i // GK
        sl = pl.ds((i % GK) * 16, 16)
        pv[row, sl] = 2 * pv[row, sl] + c
        return 0

    lax.fori_loop(0, NR * GK, cvt, 0)
    zeros16 = jnp.zeros((16,), jnp.float32)
    ones16 = jnp.full((16,), 1.0, jnp.float32)
    for i in range(G):
        ohv[i, :] = zeros16
    plsc.subcore_barrier()
    iota16 = lax.iota(jnp.int32, 16)

    def g_body(g, _):
        for k in range(GK):
            pk = pv[g, pl.ds(k * 16, 16)]
            plsc.store_scatter(ohv, [iota16 + k * 16, pk], ones16)
        pltpu.sync_copy(ohv, cnt_sp.at[dstv.at[g]], add=True)
        for k in range(GK):
            pk = pv[g, pl.ds(k * 16, 16)]
            plsc.store_scatter(ohv, [iota16 + k * 16, pk], zeros16)
        return 0

    lax.fori_loop(0, NR, g_body, 0)
    plsc.subcore_barrier()
    pltpu.sync_copy(cnt_sp.at[pl.ds(s * NPT, NPT)],
                    out_hbm.at[c, pl.ds(s * NPT, NPT)])


PASSES = 5
RPP = NR // PASSES
BPP = RPP // RING


@functools.partial(
    pl.kernel,
    out_type=jax.ShapeDtypeStruct((NC, N, D), jnp.float32),
    mesh=_mesh,
    scratch_types=[
        pltpu.VMEM((RPP, G), jnp.int32),
        pltpu.VMEM((RPP, G), jnp.int32),
        pltpu.VMEM((RING, G, P), jnp.float32),
        pltpu.VMEM((RING, G, D), jnp.float32),
        pltpu.VMEM((G + 16,), jnp.float32),
        pltpu.VMEM_SHARED((N, D), jnp.float32),
        pltpu.SemaphoreType.DMA((RING,)),
        pltpu.SemaphoreType.DMA((RING,)),
    ],
    compiler_params=_sc_params,
)
def _sc_edge(t_hbm, beta_hbm, tg_hbm, ei_hbm, z_hbm, out_hbm,
             dstv, tgv, brows, trows, sv, acc_sp, sem_t, sem_b):
    c = lax.axis_index("c")
    s = lax.axis_index("s")
    pltpu.sync_copy(z_hbm.at[pl.ds(s * NPT, NPT)], acc_sp.at[pl.ds(s * NPT, NPT)])
    plsc.subcore_barrier()
    iota16 = lax.iota(jnp.int32, 16)

    def fire(g, slot):
        @pl.when(g < RPP)
        def _():
            pltpu.make_async_copy(t_hbm.at[tgv.at[g]], trows.at[slot],
                                  sem_t.at[slot]).start()
            pltpu.make_async_copy(beta_hbm.at[dstv.at[g]], brows.at[slot],
                                  sem_b.at[slot]).start()

    for ps in range(PASSES):
        row0 = s * NR + ps * RPP
        pltpu.sync_copy(ei_hbm.at[c, pl.ds(row0, RPP)], dstv)
        pltpu.sync_copy(tg_hbm.at[c, pl.ds(row0, RPP)], tgv)
        for g0 in range(AHEAD):
            fire(jnp.int32(g0), g0)

        def blk_body(blk, _):
            for b in range(RING):
                g = blk * RING + b
                fire(g + AHEAD, (b + AHEAD) % RING)
                pltpu.make_async_copy(t_hbm.at[tgv.at[0]], trows.at[b],
                                      sem_t.at[b]).wait()
                pltpu.make_async_copy(beta_hbm.at[dstv.at[0]], brows.at[b],
                                      sem_b.at[b]).wait()
                for k in range(GK):
                    pk = tgv[g, pl.ds(k * 16, 16)] & (P - 1)
                    sv[pl.ds(k * 16, 16)] = plsc.load_gather(
                        brows.at[b], [iota16 + k * 16, pk])

                def scale_body(j, _):
                    val = sv[pl.ds(j, 16)][0]
                    for ci in range(D // 16):
                        sl = pl.ds(ci * 16, 16)
                        trows[b, j, sl] = trows[b, j, sl] * val
                    return 0

                lax.fori_loop(0, G, scale_body, 0)
                pltpu.sync_copy(trows.at[b], acc_sp.at[dstv.at[g]], add=True)
            return 0

        lax.fori_loop(0, BPP, blk_body, 0)

    plsc.subcore_barrier()
    pltpu.sync_copy(acc_sp.at[pl.ds(s * NPT, NPT)],
                    out_hbm.at[c, pl.ds(s * NPT, NPT)])


_RB = 1000
_GRID = N // _RB


def _beta_body(cnt_ref, tri_ref, beta_ref, bc_ref):
    cnt = cnt_ref[0] + cnt_ref[1]
    f = 2.0 / jnp.maximum(cnt, 1.0)
    beta = jnp.exp(jnp.dot(jnp.log(f), tri_ref[...],
                           preferred_element_type=jnp.float32))
    beta_ref[...] = beta
    bc_ref[...] = beta * cnt


def _matmul_body(x_ref, w_ref, o_ref):
    o_ref[...] = jnp.dot(x_ref[...], w_ref[...],
                         preferred_element_type=jnp.float32)


def _combine_body(x_ref, ws_ref, bs_ref, beta_ref, bc_ref, bm_ref, acc_ref,
                  o_ref, *, relu):
    h = jnp.dot(x_ref[...], ws_ref[...],
                preferred_element_type=jnp.float32) + bs_ref[...]
    h = beta_ref[:, 0:1] * h
    h = h + acc_ref[0] + acc_ref[1]
    h = h + jnp.dot(bc_ref[...], bm_ref[...],
                    preferred_element_type=jnp.float32)
    if relu:
        h = jnp.maximum(h, 0.0)
    o_ref[...] = h


def _beta_call(cnt2, tri):
    return pl.pallas_call(
        _beta_body,
        grid=(_GRID,),
        in_specs=[
            pl.BlockSpec((NC, _RB, P), lambda i: (0, i, 0)),
            pl.BlockSpec((P, P), lambda i: (0, 0)),
        ],
        out_specs=[
            pl.BlockSpec((_RB, P), lambda i: (i, 0)),
            pl.BlockSpec((_RB, P), lambda i: (i, 0)),
        ],
        out_shape=[
            jax.ShapeDtypeStruct((N, P), jnp.float32),
            jax.ShapeDtypeStruct((N, P), jnp.float32),
        ],
    )(cnt2, tri)


def _matmul_call(x, w):
    k = w.shape[1]
    return pl.pallas_call(
        _matmul_body,
        grid=(_GRID,),
        in_specs=[
            pl.BlockSpec((_RB, D), lambda i: (i, 0)),
            pl.BlockSpec((D, k), lambda i: (0, 0)),
        ],
        out_specs=pl.BlockSpec((_RB, k), lambda i: (i, 0)),
        out_shape=jax.ShapeDtypeStruct((N, k), jnp.float32),
    )(x, w)


def _combine_call(x, wself, bself, beta, bc, bmat, acc2, relu):
    return pl.pallas_call(
        functools.partial(_combine_body, relu=relu),
        grid=(_GRID,),
        in_specs=[
            pl.BlockSpec((_RB, D), lambda i: (i, 0)),
            pl.BlockSpec((D, D), lambda i: (0, 0)),
            pl.BlockSpec((1, D), lambda i: (0, 0)),
            pl.BlockSpec((_RB, P), lambda i: (i, 0)),
            pl.BlockSpec((_RB, P), lambda i: (i, 0)),
            pl.BlockSpec((P, D), lambda i: (0, 0)),
            pl.BlockSpec((NC, _RB, D), lambda i: (0, i, 0)),
        ],
        out_specs=pl.BlockSpec((_RB, D), lambda i: (i, 0)),
        out_shape=jax.ShapeDtypeStruct((N, D), jnp.float32),
    )(x, wself, bself, beta, bc, bmat, acc2)


def kernel(edge_index, edge_type, embeddings, rgcn_weights, rgcn_biases):
    ei = edge_index.astype(jnp.int32)
    et = edge_type.astype(jnp.int32)
    ei_r = ei.reshape(2, E // G, G)
    et_r = et.reshape(E // G, G)
    tg_r = (ei[::-1] * P + 2 * et[None, :]
            + jnp.array([[0], [1]], jnp.int32)).reshape(2, E // G, G)
    z16 = jnp.zeros((N, P), jnp.float32)
    z128 = jnp.zeros((N, D), jnp.float32)
    tri = jnp.triu(jnp.ones((P, P), jnp.float32)).T

    cnt2 = _sc_counts(ei_r, et_r, z16)
    beta, bc = _beta_call(cnt2, tri)

    perm = jnp.array([(q // 2) + NREL * (q % 2) for q in range(P)])
    output = embeddings
    for l in range(2):
        wall = jnp.transpose(rgcn_weights[l][perm], (1, 0, 2)).reshape(D, P * D)
        bmat = rgcn_biases[l][perm]
        t = _matmul_call(output, wall).reshape(N * P, D)
        acc2 = _sc_edge(t, beta, tg_r, ei_r, z128)
        output = _combine_call(output, rgcn_weights[l, -1],
                               rgcn_biases[l, -1].reshape(1, D),
                               beta, bc, bmat, acc2, relu=(l == 0))
    return output


# speedup vs baseline: 18.7304x; 1.1863x over previous
import functools

import jax
import jax.numpy as jnp
from jax import lax
from jax.experimental import pallas as pl
from jax.experimental.pallas import tpu as pltpu
from jax.experimental.pallas import tpu_sc as plsc

N = 10000
D = 128
NREL = 8
P = 2 * NREL
E = 320000
EI = 2 * E

NC = 2
NS = 16
NW = NC * NS
G = 32
NPT = N // NS

_mesh = plsc.VectorSubcoreMesh(core_axis_name="c", subcore_axis_name="s")
_sc_params = pltpu.CompilerParams(use_tc_tiling_on_sc=False,
                                  needs_layout_passes=False)


EPC = E // NS
NR = EPC // G
RING = 5
AHEAD = 3
SB = 25
NSB = NR // SB
GK = G // 16


@functools.partial(
    pl.kernel,
    out_type=jax.ShapeDtypeStruct((NC, N, P), jnp.float32),
    mesh=_mesh,
    scratch_types=[
        pltpu.VMEM((NR, G), jnp.int32),
        pltpu.VMEM((NR, G), jnp.int32),
        pltpu.VMEM((G, P), jnp.float32),
        pltpu.VMEM_SHARED((N, P), jnp.float32),
    ],
    compiler_params=_sc_params,
)
def _sc_counts(ei_hbm, et_hbm, z_hbm, out_hbm, dstv, pv, ohv, cnt_sp):
    c = lax.axis_index("c")
    s = lax.axis_index("s")
    pltpu.sync_copy(z_hbm.at[pl.ds(s * NPT, NPT)], cnt_sp.at[pl.ds(s * NPT, NPT)])
    pltpu.sync_copy(ei_hbm.at[c, pl.ds(s * NR, NR)], dstv)
    pltpu.sync_copy(et_hbm.at[pl.ds(s * NR, NR)], pv)

    def cvt(i, _):
        row = i // GK
        sl = pl.ds((i % GK) * 16, 16)
        pv[row, sl] = 2 * pv[row, sl] + c
        return 0

    lax.fori_loop(0, NR * GK, cvt, 0)
    zeros16 = jnp.zeros((16,), jnp.float32)
    ones16 = jnp.full((16,), 1.0, jnp.float32)
    for i in range(G):
        ohv[i, :] = zeros16
    plsc.subcore_barrier()
    iota16 = lax.iota(jnp.int32, 16)

    def g_body(g, _):
        for k in range(GK):
            pk = pv[g, pl.ds(k * 16, 16)]
            plsc.store_scatter(ohv, [iota16 + k * 16, pk], ones16)
        pltpu.sync_copy(ohv, cnt_sp.at[dstv.at[g]], add=True)
        for k in range(GK):
            pk = pv[g, pl.ds(k * 16, 16)]
            plsc.store_scatter(ohv, [iota16 + k * 16, pk], zeros16)
        return 0

    lax.fori_loop(0, NR, g_body, 0)
    plsc.subcore_barrier()
    pltpu.sync_copy(cnt_sp.at[pl.ds(s * NPT, NPT)],
                    out_hbm.at[c, pl.ds(s * NPT, NPT)])


PASSES = 5
RPP = NR // PASSES
BPP = RPP // RING


@functools.partial(
    pl.kernel,
    out_type=jax.ShapeDtypeStruct((NC, N, D), jnp.float32),
    mesh=_mesh,
    scratch_types=[
        pltpu.VMEM((RPP, G), jnp.int32),
        pltpu.VMEM((RPP, G), jnp.int32),
        pltpu.VMEM((RING, G, P), jnp.float32),
        pltpu.VMEM((RING, G, D), jnp.float32),
        pltpu.VMEM((G + 16,), jnp.float32),
        pltpu.VMEM_SHARED((N, D), jnp.float32),
        pltpu.SemaphoreType.DMA((RING,)),
        pltpu.SemaphoreType.DMA((RING,)),
        pltpu.SemaphoreType.DMA((RING,)),
    ],
    compiler_params=_sc_params,
)
def _sc_edge(t_hbm, beta_hbm, tg_hbm, ei_hbm, z_hbm, out_hbm,
             dstv, tgv, brows, trows, sv, acc_sp, sem_t, sem_b, sem_s):
    c = lax.axis_index("c")
    s = lax.axis_index("s")
    pltpu.sync_copy(z_hbm.at[pl.ds(s * NPT, NPT)], acc_sp.at[pl.ds(s * NPT, NPT)])
    plsc.subcore_barrier()
    iota16 = lax.iota(jnp.int32, 16)

    def drain_scatter(slot):
        pltpu.make_async_copy(t_hbm.at[pl.ds(0, G)], trows.at[slot],
                              sem_s.at[slot]).wait()

    def fire(g, slot):
        @pl.when(g < RPP)
        def _():
            @pl.when(g >= RING)
            def _():
                drain_scatter(slot)
            pltpu.make_async_copy(t_hbm.at[tgv.at[g]], trows.at[slot],
                                  sem_t.at[slot]).start()
            pltpu.make_async_copy(beta_hbm.at[dstv.at[g]], brows.at[slot],
                                  sem_b.at[slot]).start()

    for ps in range(PASSES):
        row0 = s * NR + ps * RPP
        pltpu.sync_copy(ei_hbm.at[c, pl.ds(row0, RPP)], dstv)
        pltpu.sync_copy(tg_hbm.at[c, pl.ds(row0, RPP)], tgv)
        for g0 in range(AHEAD):
            fire(jnp.int32(g0), g0)

        def blk_body(blk, _):
            for b in range(RING):
                g = blk * RING + b
                fire(g + AHEAD, (b + AHEAD) % RING)
                pltpu.make_async_copy(t_hbm.at[tgv.at[0]], trows.at[b],
                                      sem_t.at[b]).wait()
                pltpu.make_async_copy(beta_hbm.at[dstv.at[0]], brows.at[b],
                                      sem_b.at[b]).wait()
                for k in range(GK):
                    pk = tgv[g, pl.ds(k * 16, 16)] & (P - 1)
                    sv[pl.ds(k * 16, 16)] = plsc.load_gather(
                        brows.at[b], [iota16 + k * 16, pk])

                def scale_body(jj, _):
                    for dj in range(4):
                        j = jj * 4 + dj
                        val = sv[pl.ds(j, 16)][0]
                        for ci in range(D // 16):
                            sl = pl.ds(ci * 16, 16)
                            trows[b, j, sl] = trows[b, j, sl] * val
                    return 0

                lax.fori_loop(0, G // 4, scale_body, 0)
                pltpu.async_copy(trows.at[b], acc_sp.at[dstv.at[g]],
                                 sem_s.at[b], add=True)
            return 0

        lax.fori_loop(0, BPP, blk_body, 0)
        for b in range(RING):
            drain_scatter(b)

    plsc.subcore_barrier()
    pltpu.sync_copy(acc_sp.at[pl.ds(s * NPT, NPT)],
                    out_hbm.at[c, pl.ds(s * NPT, NPT)])


_RB = 1000
_GRID = N // _RB


def _beta_body(cnt_ref, tri_ref, beta_ref, bc_ref):
    cnt = cnt_ref[0] + cnt_ref[1]
    f = 2.0 / jnp.maximum(cnt, 1.0)
    beta = jnp.exp(jnp.dot(jnp.log(f), tri_ref[...],
                           preferred_element_type=jnp.float32))
    beta_ref[...] = beta
    bc_ref[...] = beta * cnt


def _matmul_body(x_ref, w_ref, o_ref):
    o_ref[...] = jnp.dot(x_ref[...], w_ref[...],
                         preferred_element_type=jnp.float32)


def _combine_body(x_ref, ws_ref, bs_ref, beta_ref, bc_ref, bm_ref, acc_ref,
                  o_ref, *, relu):
    h = jnp.dot(x_ref[...], ws_ref[...],
                preferred_element_type=jnp.float32) + bs_ref[...]
    h = beta_ref[:, 0:1] * h
    h = h + acc_ref[0] + acc_ref[1]
    h = h + jnp.dot(bc_ref[...], bm_ref[...],
                    preferred_element_type=jnp.float32)
    if relu:
        h = jnp.maximum(h, 0.0)
    o_ref[...] = h


def _beta_call(cnt2, tri):
    return pl.pallas_call(
        _beta_body,
        grid=(_GRID,),
        in_specs=[
            pl.BlockSpec((NC, _RB, P), lambda i: (0, i, 0)),
            pl.BlockSpec((P, P), lambda i: (0, 0)),
        ],
        out_specs=[
            pl.BlockSpec((_RB, P), lambda i: (i, 0)),
            pl.BlockSpec((_RB, P), lambda i: (i, 0)),
        ],
        out_shape=[
            jax.ShapeDtypeStruct((N, P), jnp.float32),
            jax.ShapeDtypeStruct((N, P), jnp.float32),
        ],
    )(cnt2, tri)


def _matmul_call(x, w):
    k = w.shape[1]
    return pl.pallas_call(
        _matmul_body,
        grid=(_GRID,),
        in_specs=[
            pl.BlockSpec((_RB, D), lambda i: (i, 0)),
            pl.BlockSpec((D, k), lambda i: (0, 0)),
        ],
        out_specs=pl.BlockSpec((_RB, k), lambda i: (i, 0)),
        out_shape=jax.ShapeDtypeStruct((N, k), jnp.float32),
    )(x, w)


def _combine_call(x, wself, bself, beta, bc, bmat, acc2, relu):
    return pl.pallas_call(
        functools.partial(_combine_body, relu=relu),
        grid=(_GRID,),
        in_specs=[
            pl.BlockSpec((_RB, D), lambda i: (i, 0)),
            pl.BlockSpec((D, D), lambda i: (0, 0)),
            pl.BlockSpec((1, D), lambda i: (0, 0)),
            pl.BlockSpec((_RB, P), lambda i: (i, 0)),
            pl.BlockSpec((_RB, P), lambda i: (i, 0)),
            pl.BlockSpec((P, D), lambda i: (0, 0)),
            pl.BlockSpec((NC, _RB, D), lambda i: (0, i, 0)),
        ],
        out_specs=pl.BlockSpec((_RB, D), lambda i: (i, 0)),
        out_shape=jax.ShapeDtypeStruct((N, D), jnp.float32),
    )(x, wself, bself, beta, bc, bmat, acc2)


def kernel(edge_index, edge_type, embeddings, rgcn_weights, rgcn_biases):
    ei = edge_index.astype(jnp.int32)
    et = edge_type.astype(jnp.int32)
    ei_r = ei.reshape(2, E // G, G)
    et_r = et.reshape(E // G, G)
    tg_r = (ei[::-1] * P + 2 * et[None, :]
            + jnp.array([[0], [1]], jnp.int32)).reshape(2, E // G, G)
    z16 = jnp.zeros((N, P), jnp.float32)
    z128 = jnp.zeros((N, D), jnp.float32)
    tri = jnp.triu(jnp.ones((P, P), jnp.float32)).T

    cnt2 = _sc_counts(ei_r, et_r, z16)
    beta, bc = _beta_call(cnt2, tri)

    perm = jnp.array([(q // 2) + NREL * (q % 2) for q in range(P)])
    output = embeddings
    for l in range(2):
        wall = jnp.transpose(rgcn_weights[l][perm], (1, 0, 2)).reshape(D, P * D)
        bmat = rgcn_biases[l][perm]
        t = _matmul_call(output, wall).reshape(N * P, D)
        acc2 = _sc_edge(t, beta, tg_r, ei_r, z128)
        output = _combine_call(output, rgcn_weights[l, -1],
                               rgcn_biases[l, -1].reshape(1, D),
                               beta, bc, bmat, acc2, relu=(l == 0))
    return output
